# baseline (speedup 1.0000x reference)
import jax
import jax.numpy as jnp
from jax import lax
from jax.experimental import pallas as pl
from jax.experimental.pallas import tpu as pltpu
from jax.experimental.pallas import tpu_sc as plsc

N_NODES = 10000
E_EDGES = 320000
D = 128
C_OUT = 8
B_GRAPHS = 64
NSEG = 3 * B_GRAPHS
SEGP = 256

NTILES = 32
NP = 10240
CH = 48
NBUF = 6
EPAD = 331776
EPT = EPAD // NTILES
NCH = EPT // CH
RPT = NP // NTILES
CSEG = 64
APT = NP // 16


def _mesh():
    return plsc.VectorSubcoreMesh(core_axis_name="c", subcore_axis_name="s")


_SC_PARAMS = pltpu.CompilerParams(use_tc_tiling_on_sc=False)
_SC_PARAMS_NL = pltpu.CompilerParams(
    use_tc_tiling_on_sc=False, needs_layout_passes=False
)


def _zero_rows(rows, nrows):
    @pl.loop(0, nrows)
    def _(r):
        for k in range(D // 16):
            rows[r, pl.ds(k * 16, 16)] = jnp.zeros((16,), jnp.float32)


_DEG_DEPTH = 8


def _deg_body(d_hbm, out_hbm, dref, ones_v, zv, acc, semS):
    c = lax.axis_index("c")
    s = lax.axis_index("s")
    gw = c * 16 + s
    for k in range(CH // 16):
        ones_v[pl.ds(k * 16, 16)] = jnp.ones((16,), jnp.float32)
    for k in range(APT // 16):
        zv[pl.ds(k * 16, 16)] = jnp.zeros((16,), jnp.float32)
    pltpu.sync_copy(zv, acc.at[pl.ds(s * APT, APT)])
    pltpu.sync_copy(d_hbm.at[pl.ds(gw * NCH, NCH)], dref)
    plsc.subcore_barrier()

    def wait_one():
        pltpu.make_async_copy(ones_v, acc.at[dref.at[0]], semS).wait()

    @pl.loop(0, NCH)
    def _(j):
        pltpu.async_copy(ones_v, acc.at[dref.at[j]], semS, add=True)

        @pl.when(j >= _DEG_DEPTH)
        def _():
            wait_one()

    @pl.loop(0, _DEG_DEPTH)
    def _(j):
        wait_one()

    plsc.subcore_barrier()
    pltpu.sync_copy(acc.at[pl.ds(s * APT, APT)], out_hbm.at[c, s])


def _degree(d2):
    k = pl.kernel(
        _deg_body,
        out_type=jax.ShapeDtypeStruct((2, 16, APT), jnp.float32),
        mesh=_mesh(),
        compiler_params=_SC_PARAMS,
        scratch_types=[
            pltpu.VMEM((NCH, CH), jnp.int32),
            pltpu.VMEM((CH,), jnp.float32),
            pltpu.VMEM((APT,), jnp.float32),
            pltpu.VMEM_SHARED((NP,), jnp.float32),
            pltpu.SemaphoreType.DMA,
        ],
    )
    return k(d2)


def _msg_body(q_hbm, s_hbm, d_hbm, out_hbm, *refs):
    rows = list(refs[2:2 + NBUF])
    sref, dref = refs[0], refs[1]
    acc = refs[2 + NBUF]
    semG = list(refs[3 + NBUF:3 + 2 * NBUF])
    semS = list(refs[3 + 2 * NBUF:3 + 3 * NBUF])
    semD = list(refs[3 + 3 * NBUF:3 + 4 * NBUF])
    c = lax.axis_index("c")
    s = lax.axis_index("s")
    gw = c * 16 + s
    pltpu.sync_copy(s_hbm.at[pl.ds(gw * NCH, NCH)], sref.at[pl.ds(0, NCH)])
    for r in range(NBUF):
        for k in range(CH // 16):
            sref[NCH + r, pl.ds(k * 16, 16)] = lax.iota(jnp.int32, 16) + (16 * k)

    def gath(i, j):
        pltpu.async_copy(q_hbm.at[sref.at[j]], rows[i], semG[i])

    def scat(i):
        pltpu.async_copy(rows[i], acc.at[dref.at[i]], semS[i], add=True)

    def pref_d(i, j):
        pltpu.async_copy(d_hbm.at[gw * NCH + j], dref.at[i], semD[i])

    def wait_g(i):
        pltpu.make_async_copy(q_hbm.at[sref.at[0]], rows[i], semG[i]).wait()

    def wait_s(i):
        pltpu.make_async_copy(rows[i], acc.at[dref.at[0]], semS[i]).wait()

    def wait_d(i):
        pltpu.make_async_copy(d_hbm.at[0], dref.at[i], semD[i]).wait()

    for i in range(NBUF):
        pref_d(i, i)
        if i >= 1:
            gath(i, i)
    _zero_rows(rows[0], CH)
    for j in range(APT // CH):
        pltpu.sync_copy(rows[0], acc.at[pl.ds(s * APT + j * CH, CH)])
    _ZREM = APT - (APT // CH) * CH
    if _ZREM:
        pltpu.sync_copy(rows[0].at[pl.ds(0, _ZREM)],
                        acc.at[pl.ds(s * APT + (APT // CH) * CH, _ZREM)])
    plsc.subcore_barrier()
    gath(0, 0)

    @pl.loop(0, NCH, step=NBUF)
    def _(j):
        for i in range(NBUF):
            wait_g(i)
            wait_d(i)
            scat(i)
            if i >= 1:
                wait_s(i - 1)
                pref_d(i - 1, j + NBUF + i - 1)
                gath(i - 1, j + NBUF + i - 1)
        wait_s(NBUF - 1)
        pref_d(NBUF - 1, j + 2 * NBUF - 1)
        gath(NBUF - 1, j + 2 * NBUF - 1)

    for i in range(NBUF):
        wait_g(i)
        wait_d(i)
    plsc.subcore_barrier()
    pltpu.sync_copy(acc.at[pl.ds(s * APT, APT)], out_hbm.at[c, s])


def _message(q, s2, d2):
    k = pl.kernel(
        _msg_body,
        out_type=jax.ShapeDtypeStruct((2, 16, APT, D), jnp.float32),
        mesh=_mesh(),
        compiler_params=_SC_PARAMS,
        scratch_types=[
            pltpu.VMEM((NCH + NBUF, CH), jnp.int32),
            pltpu.VMEM((NBUF, CH), jnp.int32),
        ] + [pltpu.VMEM((CH, D), jnp.float32)] * NBUF + [
            pltpu.VMEM_SHARED((NP, D), jnp.float32),
        ] + [pltpu.SemaphoreType.DMA] * (3 * NBUF),
    )
    return k(q, s2, d2).reshape(2, NP, D)


def _seg_body(a_hbm, cb_hbm, dinv_hbm, sums_hbm, cnts_hbm,
              a0v, a1v, cbv2, dvv, buf, onesr, zc, acc, cacc, sem):
    c = lax.axis_index("c")
    s = lax.axis_index("s")
    gw = c * 16 + s
    _zero_rows(buf, 16)
    @pl.loop(0, CSEG)
    def _(r):
        onesr[r, pl.ds(0, 16)] = jnp.ones((16,), jnp.float32)
    @pl.loop(0, 16)
    def _(r):
        zc[r, pl.ds(0, 16)] = jnp.zeros((16,), jnp.float32)
    pltpu.sync_copy(buf.at[pl.ds(0, 16)], acc.at[pl.ds(s * (SEGP // 16), SEGP // 16)])
    pltpu.sync_copy(zc, cacc.at[pl.ds(s * (SEGP // 16), SEGP // 16)])
    pltpu.sync_copy(a_hbm.at[0, pl.ds(gw * RPT, RPT)], a0v)
    pltpu.sync_copy(a_hbm.at[1, pl.ds(gw * RPT, RPT)], a1v)
    pltpu.sync_copy(cb_hbm.at[pl.ds(gw * (RPT // CSEG), RPT // CSEG)], cbv2)
    pltpu.sync_copy(dinv_hbm.at[pl.ds(gw * RPT, RPT)], dvv)
    plsc.subcore_barrier()

    for j in range(RPT // CSEG):
        @pl.loop(0, CSEG)
        def _(r):
            row = j * CSEG + r
            dv = plsc.load_gather(dvv, [jnp.full((16,), row, jnp.int32)])
            for k in range(D // 16):
                sl = pl.ds(k * 16, 16)
                buf[r, sl] = (a0v[row, sl] + a1v[row, sl]) * dv
        pltpu.sync_copy(buf, acc.at[cbv2.at[j]], add=True)
        pltpu.sync_copy(onesr, cacc.at[cbv2.at[j]], add=True)

    plsc.subcore_barrier()
    pltpu.sync_copy(acc.at[pl.ds(s * (SEGP // 16), SEGP // 16)], sums_hbm.at[c, s])
    pltpu.sync_copy(cacc.at[pl.ds(s * (SEGP // 16), SEGP // 16)], cnts_hbm.at[c, s])


def _segment(a3, cb2, dinv):
    k = pl.kernel(
        _seg_body,
        out_type=[
            jax.ShapeDtypeStruct((2, 16, SEGP // 16, D), jnp.float32),
            jax.ShapeDtypeStruct((2, 16, SEGP // 16, 16), jnp.float32),
        ],
        mesh=_mesh(),
        compiler_params=_SC_PARAMS_NL,
        scratch_types=[
            pltpu.VMEM((RPT, D), jnp.float32),
            pltpu.VMEM((RPT, D), jnp.float32),
            pltpu.VMEM((RPT // CSEG, CSEG), jnp.int32),
            pltpu.VMEM((RPT,), jnp.float32),
            pltpu.VMEM((CSEG, D), jnp.float32),
            pltpu.VMEM((CSEG, 16), jnp.float32),
            pltpu.VMEM((16, 16), jnp.float32),
            pltpu.VMEM_SHARED((SEGP, D), jnp.float32),
            pltpu.VMEM_SHARED((SEGP, 16), jnp.float32),
            pltpu.SemaphoreType.DMA,
        ],
    )
    sums, cnts = k(a3, cb2, dinv)
    return sums.reshape(2, SEGP, D), cnts.reshape(2, SEGP, 16)


BM = 10240


def _mm1_body(x_ref, w_ref, dv_ref, o_ref):
    o_ref[...] = (
        jnp.dot(x_ref[...], w_ref[...], preferred_element_type=jnp.float32)
        * dv_ref[...]
    )


def _mm1(x_pad, W, dinv2):
    return pl.pallas_call(
        _mm1_body,
        grid=(NP // BM,),
        in_specs=[
            pl.BlockSpec((BM, D), lambda i: (i, 0)),
            pl.BlockSpec((D, D), lambda i: (0, 0)),
            pl.BlockSpec((BM, 1), lambda i: (i, 0)),
        ],
        out_specs=pl.BlockSpec((BM, D), lambda i: (i, 0)),
        out_shape=jax.ShapeDtypeStruct((NP, D), jnp.float32),
    )(x_pad, W, dinv2)


def _fused_body(a_ref, dv_ref, b_ref, w_ref, o_ref):
    m = (a_ref[0] + a_ref[1]) * dv_ref[...] + b_ref[...]
    m = jnp.maximum(m, 0.0)
    o_ref[...] = (
        jnp.dot(m, w_ref[...], preferred_element_type=jnp.float32) * dv_ref[...]
    )


def _fused(acc, dinv2, b, W):
    return pl.pallas_call(
        _fused_body,
        grid=(NP // BM,),
        in_specs=[
            pl.BlockSpec((2, BM, D), lambda i: (0, i, 0)),
            pl.BlockSpec((BM, 1), lambda i: (i, 0)),
            pl.BlockSpec((1, D), lambda i: (0, 0)),
            pl.BlockSpec((D, D), lambda i: (0, 0)),
        ],
        out_specs=pl.BlockSpec((BM, D), lambda i: (i, 0)),
        out_shape=jax.ShapeDtypeStruct((NP, D), jnp.float32),
    )(acc, dinv2, b, W)


def _ep_body(sm_ref, ct_ref, b3_ref, wl_ref, bl_ref, o_ref):
    sm = sm_ref[0] + sm_ref[1]
    ct = ct_ref[0] + ct_ref[1]
    ctc = ct[:, 0:1]
    xc = (sm + ctc * b3_ref[...]) / jnp.maximum(ctc, 1.0)
    z = (
        jnp.dot(xc[0:B_GRAPHS], wl_ref[0], preferred_element_type=jnp.float32)
        + jnp.dot(xc[B_GRAPHS:2 * B_GRAPHS], wl_ref[1],
                  preferred_element_type=jnp.float32)
        + jnp.dot(xc[2 * B_GRAPHS:3 * B_GRAPHS], wl_ref[2],
                  preferred_element_type=jnp.float32)
    )
    o_ref[...] = jax.nn.sigmoid(z + bl_ref[...])


def _epilogue(sums, cnts, b3, Wl3, bl2):
    return pl.pallas_call(
        _ep_body,
        out_shape=jax.ShapeDtypeStruct((B_GRAPHS, C_OUT), jnp.float32),
    )(sums, cnts, b3, Wl3, bl2)


def kernel(x, edge_index, batch, W1, b1, W2, b2, W3, b3, Wl, bl):
    src = edge_index[0]
    dst = edge_index[1]
    iota = jnp.arange(N_NODES, dtype=jnp.int32)
    npad_e = EPAD - E_EDGES - N_NODES
    pad_idx = N_NODES + (jnp.arange(npad_e, dtype=jnp.int32) % (NP - N_NODES))
    s2 = jnp.concatenate([src, iota, pad_idx]).reshape(EPAD // CH, CH)
    d2 = jnp.pad(
        jnp.concatenate([dst, iota, pad_idx]).reshape(EPAD // CH, CH),
        ((0, NBUF), (0, 0)),
    )
    x_pad = jnp.pad(x, ((0, NP - N_NODES), (0, 0)))

    deg = _degree(d2).reshape(2, NP).sum(axis=0)
    dinv = lax.rsqrt(jnp.maximum(deg, 1e-12))
    dinv2 = dinv[:, None]

    q1 = _mm1(x_pad, W1, dinv2)
    a1 = _message(q1, s2, d2)
    q2 = _fused(a1, dinv2, b1.reshape(1, D), W2)
    a2 = _message(q2, s2, d2)
    q3 = _fused(a2, dinv2, b2.reshape(1, D), W3)
    a3 = _message(q3, s2, d2)

    cluster = (x[:, -1] + 2.0 * x[:, -2]).astype(jnp.int32)
    cb = cluster * B_GRAPHS + batch
    cb_pad = jnp.concatenate(
        [cb, NSEG + (jnp.arange(NP - N_NODES, dtype=jnp.int32) % (SEGP - NSEG))]
    ).reshape(NP // CSEG, CSEG)
    sums, cnts = _segment(a3, cb_pad, dinv)
    return _epilogue(sums, cnts, b3.reshape(1, D), Wl.reshape(3, D, C_OUT),
                     bl.reshape(1, C_OUT))


# speedup vs baseline: 1.0048x; 1.0048x over previous
import jax
import jax.numpy as jnp
from jax import lax
from jax.experimental import pallas as pl
from jax.experimental.pallas import tpu as pltpu
from jax.experimental.pallas import tpu_sc as plsc

N_NODES = 10000
E_EDGES = 320000
D = 128
C_OUT = 8
B_GRAPHS = 64
NSEG = 3 * B_GRAPHS
SEGP = 256

NTILES = 32
NP = 10240
CH = 48
NBUF = 6
EPAD = 331776
EPT = EPAD // NTILES
NCH = EPT // CH
RPT = NP // NTILES
CSEG = 64
APT = NP // 16


def _mesh():
    return plsc.VectorSubcoreMesh(core_axis_name="c", subcore_axis_name="s")


_SC_PARAMS = pltpu.CompilerParams(use_tc_tiling_on_sc=False)
_SC_PARAMS_NL = pltpu.CompilerParams(
    use_tc_tiling_on_sc=False, needs_layout_passes=False
)


def _zero_rows(rows, nrows):
    @pl.loop(0, nrows)
    def _(r):
        for k in range(D // 16):
            rows[r, pl.ds(k * 16, 16)] = jnp.zeros((16,), jnp.float32)


_DEG_DEPTH = 8


def _deg_body(d_hbm, out_hbm, dref, ones_v, zv, acc, semS):
    c = lax.axis_index("c")
    s = lax.axis_index("s")
    gw = c * 16 + s
    for k in range(CH // 16):
        ones_v[pl.ds(k * 16, 16)] = jnp.ones((16,), jnp.float32)
    for k in range(APT // 16):
        zv[pl.ds(k * 16, 16)] = jnp.zeros((16,), jnp.float32)
    pltpu.sync_copy(zv, acc.at[pl.ds(s * APT, APT)])
    pltpu.sync_copy(d_hbm.at[pl.ds(gw * NCH, NCH)], dref)
    plsc.subcore_barrier()

    def wait_one():
        pltpu.make_async_copy(ones_v, acc.at[dref.at[0]], semS).wait()

    @pl.loop(0, NCH)
    def _(j):
        pltpu.async_copy(ones_v, acc.at[dref.at[j]], semS, add=True)

        @pl.when(j >= _DEG_DEPTH)
        def _():
            wait_one()

    @pl.loop(0, _DEG_DEPTH)
    def _(j):
        wait_one()

    plsc.subcore_barrier()
    pltpu.sync_copy(acc.at[pl.ds(s * APT, APT)], out_hbm.at[c, s])


def _degree(d2):
    k = pl.kernel(
        _deg_body,
        out_type=jax.ShapeDtypeStruct((2, 16, APT), jnp.float32),
        mesh=_mesh(),
        compiler_params=_SC_PARAMS,
        scratch_types=[
            pltpu.VMEM((NCH, CH), jnp.int32),
            pltpu.VMEM((CH,), jnp.float32),
            pltpu.VMEM((APT,), jnp.float32),
            pltpu.VMEM_SHARED((NP,), jnp.float32),
            pltpu.SemaphoreType.DMA,
        ],
    )
    return k(d2)


def _msg_body(q_hbm, s_hbm, d_hbm, out_hbm, *refs):
    rows = list(refs[2:2 + NBUF])
    sref, dref = refs[0], refs[1]
    acc = refs[2 + NBUF]
    semG = list(refs[3 + NBUF:3 + 2 * NBUF])
    semS = list(refs[3 + 2 * NBUF:3 + 3 * NBUF])
    semD = list(refs[3 + 3 * NBUF:3 + 4 * NBUF])
    c = lax.axis_index("c")
    s = lax.axis_index("s")
    gw = c * 16 + s
    pltpu.sync_copy(s_hbm.at[pl.ds(gw * NCH, NCH)], sref.at[pl.ds(0, NCH)])
    for r in range(NBUF):
        for k in range(CH // 16):
            sref[NCH + r, pl.ds(k * 16, 16)] = lax.iota(jnp.int32, 16) + (16 * k)

    def gath(i, j):
        pltpu.async_copy(q_hbm.at[sref.at[j]], rows[i], semG[i])

    def scat(i):
        pltpu.async_copy(rows[i], acc.at[dref.at[i]], semS[i], add=True)

    def pref_d(i, j):
        pltpu.async_copy(d_hbm.at[gw * NCH + j], dref.at[i], semD[i])

    def wait_g(i):
        pltpu.make_async_copy(q_hbm.at[sref.at[0]], rows[i], semG[i]).wait()

    def wait_s(i):
        pltpu.make_async_copy(rows[i], acc.at[dref.at[0]], semS[i]).wait()

    def wait_d(i):
        pltpu.make_async_copy(d_hbm.at[0], dref.at[i], semD[i]).wait()

    for i in range(NBUF):
        pref_d(i, i)
        if i >= 1:
            gath(i, i)
    _zero_rows(rows[0], CH)
    for j in range(APT // CH):
        pltpu.sync_copy(rows[0], acc.at[pl.ds(s * APT + j * CH, CH)])
    _ZREM = APT - (APT // CH) * CH
    if _ZREM:
        pltpu.sync_copy(rows[0].at[pl.ds(0, _ZREM)],
                        acc.at[pl.ds(s * APT + (APT // CH) * CH, _ZREM)])
    plsc.subcore_barrier()
    gath(0, 0)

    @pl.loop(0, NCH, step=NBUF)
    def _(j):
        for i in range(NBUF):
            wait_g(i)
            wait_d(i)
            scat(i)
            if i >= 1:
                wait_s(i - 1)
                pref_d(i - 1, j + NBUF + i - 1)
                gath(i - 1, j + NBUF + i - 1)
        wait_s(NBUF - 1)
        pref_d(NBUF - 1, j + 2 * NBUF - 1)
        gath(NBUF - 1, j + 2 * NBUF - 1)

    for i in range(NBUF):
        wait_g(i)
        wait_d(i)
    plsc.subcore_barrier()
    pltpu.sync_copy(acc.at[pl.ds(s * APT, APT)], out_hbm.at[c, s])


def _message(q, s2, d2):
    k = pl.kernel(
        _msg_body,
        out_type=jax.ShapeDtypeStruct((2, 16, APT, D), jnp.float32),
        mesh=_mesh(),
        compiler_params=_SC_PARAMS,
        scratch_types=[
            pltpu.VMEM((NCH + NBUF, CH), jnp.int32),
            pltpu.VMEM((NBUF, CH), jnp.int32),
        ] + [pltpu.VMEM((CH, D), jnp.float32)] * NBUF + [
            pltpu.VMEM_SHARED((NP, D), jnp.float32),
        ] + [pltpu.SemaphoreType.DMA] * (3 * NBUF),
    )
    return k(q, s2, d2).reshape(2, NP, D)


def _seg_body(a_hbm, cb_hbm, dinv_hbm, sums_hbm, cnts_hbm,
              a0v, a1v, cbv2, dvv, buf, onesr, zc, acc, cacc, sem):
    c = lax.axis_index("c")
    s = lax.axis_index("s")
    gw = c * 16 + s
    _zero_rows(buf, 16)
    @pl.loop(0, CSEG)
    def _(r):
        onesr[r, pl.ds(0, 16)] = jnp.ones((16,), jnp.float32)
    @pl.loop(0, 16)
    def _(r):
        zc[r, pl.ds(0, 16)] = jnp.zeros((16,), jnp.float32)
    pltpu.sync_copy(buf.at[pl.ds(0, 16)], acc.at[pl.ds(s * (SEGP // 16), SEGP // 16)])
    pltpu.sync_copy(zc, cacc.at[pl.ds(s * (SEGP // 16), SEGP // 16)])
    pltpu.sync_copy(a_hbm.at[0, pl.ds(gw * RPT, RPT)], a0v)
    pltpu.sync_copy(a_hbm.at[1, pl.ds(gw * RPT, RPT)], a1v)
    pltpu.sync_copy(cb_hbm.at[pl.ds(gw * (RPT // CSEG), RPT // CSEG)], cbv2)
    pltpu.sync_copy(dinv_hbm.at[pl.ds(gw * RPT, RPT)], dvv)
    plsc.subcore_barrier()

    for j in range(RPT // CSEG):
        @pl.loop(0, CSEG)
        def _(r):
            row = j * CSEG + r
            dv = plsc.load_gather(dvv, [jnp.full((16,), row, jnp.int32)])
            for k in range(D // 16):
                sl = pl.ds(k * 16, 16)
                buf[r, sl] = (a0v[row, sl] + a1v[row, sl]) * dv
        pltpu.sync_copy(buf, acc.at[cbv2.at[j]], add=True)
        pltpu.sync_copy(onesr, cacc.at[cbv2.at[j]], add=True)

    plsc.subcore_barrier()
    pltpu.sync_copy(acc.at[pl.ds(s * (SEGP // 16), SEGP // 16)], sums_hbm.at[c, s])
    pltpu.sync_copy(cacc.at[pl.ds(s * (SEGP // 16), SEGP // 16)], cnts_hbm.at[c, s])


def _segment(a3, cb2, dinv):
    k = pl.kernel(
        _seg_body,
        out_type=[
            jax.ShapeDtypeStruct((2, 16, SEGP // 16, D), jnp.float32),
            jax.ShapeDtypeStruct((2, 16, SEGP // 16, 16), jnp.float32),
        ],
        mesh=_mesh(),
        compiler_params=_SC_PARAMS_NL,
        scratch_types=[
            pltpu.VMEM((RPT, D), jnp.float32),
            pltpu.VMEM((RPT, D), jnp.float32),
            pltpu.VMEM((RPT // CSEG, CSEG), jnp.int32),
            pltpu.VMEM((RPT,), jnp.float32),
            pltpu.VMEM((CSEG, D), jnp.float32),
            pltpu.VMEM((CSEG, 16), jnp.float32),
            pltpu.VMEM((16, 16), jnp.float32),
            pltpu.VMEM_SHARED((SEGP, D), jnp.float32),
            pltpu.VMEM_SHARED((SEGP, 16), jnp.float32),
            pltpu.SemaphoreType.DMA,
        ],
    )
    sums, cnts = k(a3, cb2, dinv)
    return sums.reshape(2, SEGP, D), cnts.reshape(2, SEGP, 16)


BM = 5120


def _mm1_body(x_ref, w_ref, dv_ref, o_ref):
    o_ref[...] = (
        jnp.dot(x_ref[...], w_ref[...], preferred_element_type=jnp.float32)
        * dv_ref[...]
    )


def _mm1(x_pad, W, dinv2):
    return pl.pallas_call(
        _mm1_body,
        grid=(NP // BM,),
        in_specs=[
            pl.BlockSpec((BM, D), lambda i: (i, 0)),
            pl.BlockSpec((D, D), lambda i: (0, 0)),
            pl.BlockSpec((BM, 1), lambda i: (i, 0)),
        ],
        out_specs=pl.BlockSpec((BM, D), lambda i: (i, 0)),
        out_shape=jax.ShapeDtypeStruct((NP, D), jnp.float32),
    )(x_pad, W, dinv2)


def _fused_body(a_ref, dv_ref, b_ref, w_ref, o_ref):
    m = (a_ref[0] + a_ref[1]) * dv_ref[...] + b_ref[...]
    m = jnp.maximum(m, 0.0)
    o_ref[...] = (
        jnp.dot(m, w_ref[...], preferred_element_type=jnp.float32) * dv_ref[...]
    )


def _fused(acc, dinv2, b, W):
    return pl.pallas_call(
        _fused_body,
        grid=(NP // BM,),
        in_specs=[
            pl.BlockSpec((2, BM, D), lambda i: (0, i, 0)),
            pl.BlockSpec((BM, 1), lambda i: (i, 0)),
            pl.BlockSpec((1, D), lambda i: (0, 0)),
            pl.BlockSpec((D, D), lambda i: (0, 0)),
        ],
        out_specs=pl.BlockSpec((BM, D), lambda i: (i, 0)),
        out_shape=jax.ShapeDtypeStruct((NP, D), jnp.float32),
    )(acc, dinv2, b, W)


def _ep_body(sm_ref, ct_ref, b3_ref, wl_ref, bl_ref, o_ref):
    sm = sm_ref[0] + sm_ref[1]
    ct = ct_ref[0] + ct_ref[1]
    ctc = ct[:, 0:1]
    xc = (sm + ctc * b3_ref[...]) / jnp.maximum(ctc, 1.0)
    z = (
        jnp.dot(xc[0:B_GRAPHS], wl_ref[0], preferred_element_type=jnp.float32)
        + jnp.dot(xc[B_GRAPHS:2 * B_GRAPHS], wl_ref[1],
                  preferred_element_type=jnp.float32)
        + jnp.dot(xc[2 * B_GRAPHS:3 * B_GRAPHS], wl_ref[2],
                  preferred_element_type=jnp.float32)
    )
    o_ref[...] = jax.nn.sigmoid(z + bl_ref[...])


def _epilogue(sums, cnts, b3, Wl3, bl2):
    return pl.pallas_call(
        _ep_body,
        out_shape=jax.ShapeDtypeStruct((B_GRAPHS, C_OUT), jnp.float32),
    )(sums, cnts, b3, Wl3, bl2)


def kernel(x, edge_index, batch, W1, b1, W2, b2, W3, b3, Wl, bl):
    src = edge_index[0]
    dst = edge_index[1]
    iota = jnp.arange(N_NODES, dtype=jnp.int32)
    npad_e = EPAD - E_EDGES - N_NODES
    pad_idx = N_NODES + (jnp.arange(npad_e, dtype=jnp.int32) % (NP - N_NODES))
    s2 = jnp.concatenate([src, iota, pad_idx]).reshape(EPAD // CH, CH)
    d2 = jnp.pad(
        jnp.concatenate([dst, iota, pad_idx]).reshape(EPAD // CH, CH),
        ((0, NBUF), (0, 0)),
    )
    x_pad = jnp.pad(x, ((0, NP - N_NODES), (0, 0)))

    deg = _degree(d2).reshape(2, NP).sum(axis=0)
    dinv = lax.rsqrt(jnp.maximum(deg, 1e-12))
    dinv2 = dinv[:, None]

    q1 = _mm1(x_pad, W1, dinv2)
    a1 = _message(q1, s2, d2)
    q2 = _fused(a1, dinv2, b1.reshape(1, D), W2)
    a2 = _message(q2, s2, d2)
    q3 = _fused(a2, dinv2, b2.reshape(1, D), W3)
    a3 = _message(q3, s2, d2)

    cluster = (x[:, -1] + 2.0 * x[:, -2]).astype(jnp.int32)
    cb = cluster * B_GRAPHS + batch
    cb_pad = jnp.concatenate(
        [cb, NSEG + (jnp.arange(NP - N_NODES, dtype=jnp.int32) % (SEGP - NSEG))]
    ).reshape(NP // CSEG, CSEG)
    sums, cnts = _segment(a3, cb_pad, dinv)
    return _epilogue(sums, cnts, b3.reshape(1, D), Wl.reshape(3, D, C_OUT),
                     bl.reshape(1, C_OUT))
